# Initial kernel scaffold; baseline (speedup 1.0000x reference)
import jax
import jax.numpy as jnp
from jax.experimental import pallas as pl


def kernel(x, table):
    raise NotImplementedError("write your pallas kernel here")


# speedup vs baseline: 1.7144x; 1.7144x over previous
import functools

import jax
import jax.numpy as jnp
from jax import lax
from jax.experimental import pallas as pl
from jax.experimental.pallas import tpu as pltpu
from jax.experimental.pallas import tpu_sc as plsc

L = 16
D = 64
G = 128
CHUNK = 512
GPC = CHUNK // G


def _lane_shuffle(x, perm):
    dnums = lax.GatherDimensionNumbers(
        offset_dims=(), collapsed_slice_dims=(0,), start_index_map=(0,))
    return lax.gather(x, perm[:, None], dnums, slice_sizes=(1,),
                      mode=lax.GatherScatterMode.PROMISE_IN_BOUNDS)


def _normalize_chunk(rows_v, buf):
    iota = lax.iota(jnp.int32, L)
    perms = [iota ^ sh for sh in (8, 4, 2, 1)]

    def row_body(r, _):
        v = [rows_v[buf, r, pl.ds(i * L, L)] for i in range(D // L)]
        s = v[0] * v[0]
        for i in range(1, D // L):
            s = s + v[i] * v[i]
        for p in perms:
            s = s + _lane_shuffle(s, p)
        bits = plsc.bitcast(s, jnp.int32)
        y = plsc.bitcast(jnp.int32(0x5F3759DF) - (bits >> 1), jnp.float32)
        hs = s * jnp.float32(0.5)
        y = y * (jnp.float32(1.5) - hs * y * y)
        y = y * (jnp.float32(1.5) - hs * y * y)
        for i in range(D // L):
            rows_v[buf, r, pl.ds(i * L, L)] = v[i] * y
        return ()

    lax.fori_loop(0, CHUNK, row_body, (), unroll=4)


def _sc_embed_norm(table, idx2d, *, n_rows):
    info = plsc.get_sparse_core_info()
    nc, ns = info.num_cores, info.num_subcores
    nw = nc * ns
    per_w = n_rows // nw
    assert per_w % CHUNK == 0
    n_chunks = per_w // CHUNK
    idx_rows_per_w = per_w // G

    mesh = plsc.VectorSubcoreMesh(core_axis_name="c", subcore_axis_name="s")

    @functools.partial(
        pl.kernel,
        out_type=jax.ShapeDtypeStruct((n_rows, D), jnp.float32),
        mesh=mesh,
        scratch_types=[
            pltpu.VMEM((idx_rows_per_w, G), jnp.int32),
            pltpu.VMEM((1, CHUNK, D), jnp.float32),
            pltpu.SemaphoreType.DMA,
        ],
        compiler_params=pltpu.CompilerParams(
            needs_layout_passes=False, use_tc_tiling_on_sc=False),
    )
    def k(table_hbm, idx_hbm, out_hbm, idx_v, rows_v, gsem):
        wid = lax.axis_index("s") * nc + lax.axis_index("c")
        row_base = wid * per_w
        pltpu.sync_copy(idx_hbm.at[pl.ds(wid * idx_rows_per_w, idx_rows_per_w)],
                        idx_v)

        def chunk_body(c, _):
            for g in range(GPC):
                pltpu.async_copy(
                    table_hbm.at[idx_v.at[c * GPC + g]],
                    rows_v.at[0, pl.ds(g * G, G)],
                    gsem,
                )
            for g in range(GPC):
                pltpu.make_async_copy(
                    table_hbm.at[idx_v.at[c * GPC + g]],
                    rows_v.at[0, pl.ds(g * G, G)],
                    gsem,
                ).wait()
            _normalize_chunk(rows_v, 0)
            pltpu.sync_copy(rows_v.at[0],
                            out_hbm.at[pl.ds(row_base + c * CHUNK, CHUNK)])
            return ()

        lax.fori_loop(0, n_chunks, chunk_body, ())

    return k(table, idx2d)


def kernel(x, table):
    b, h = x.shape
    n = b * h
    idx2d = x.reshape(n // G, G).astype(jnp.int32)
    out = _sc_embed_norm(table, idx2d, n_rows=n)
    return out.reshape(b, h, D)


# speedup vs baseline: 1.9226x; 1.1215x over previous
import functools

import jax
import jax.numpy as jnp
from jax import lax
from jax.experimental import pallas as pl
from jax.experimental.pallas import tpu as pltpu
from jax.experimental.pallas import tpu_sc as plsc

L = 16
D = 64
G = 128
CHUNK = 256
GPC = CHUNK // G
NBUF = 4


def _lane_shuffle(x, perm):
    dnums = lax.GatherDimensionNumbers(
        offset_dims=(), collapsed_slice_dims=(0,), start_index_map=(0,))
    return lax.gather(x, perm[:, None], dnums, slice_sizes=(1,),
                      mode=lax.GatherScatterMode.PROMISE_IN_BOUNDS)


def _normalize_chunk(rows_v, buf):
    iota = lax.iota(jnp.int32, L)
    perms = [iota ^ sh for sh in (8, 4, 2, 1)]

    def row_body(r, _):
        v = [rows_v[buf, r, pl.ds(i * L, L)] for i in range(D // L)]
        s = v[0] * v[0]
        for i in range(1, D // L):
            s = s + v[i] * v[i]
        for p in perms:
            s = s + _lane_shuffle(s, p)
        bits = plsc.bitcast(s, jnp.int32)
        y = plsc.bitcast(jnp.int32(0x5F3759DF) - (bits >> 1), jnp.float32)
        hs = s * jnp.float32(0.5)
        y = y * (jnp.float32(1.5) - hs * y * y)
        y = y * (jnp.float32(1.5) - hs * y * y)
        for i in range(D // L):
            rows_v[buf, r, pl.ds(i * L, L)] = v[i] * y
        return ()

    lax.fori_loop(0, CHUNK, row_body, (), unroll=4)


def _sc_embed_norm(table, idx2d, *, n_rows):
    info = plsc.get_sparse_core_info()
    nc, ns = info.num_cores, info.num_subcores
    nw = nc * ns
    per_w = n_rows // nw
    assert per_w % (CHUNK * NBUF) == 0
    n_chunks = per_w // CHUNK
    idx_rows_per_w = per_w // G

    mesh = plsc.VectorSubcoreMesh(core_axis_name="c", subcore_axis_name="s")

    @functools.partial(
        pl.kernel,
        out_type=jax.ShapeDtypeStruct((n_rows, D), jnp.float32),
        mesh=mesh,
        scratch_types=[
            pltpu.VMEM((idx_rows_per_w, G), jnp.int32),
            pltpu.VMEM((NBUF, CHUNK, D), jnp.float32),
            pltpu.SemaphoreType.DMA((NBUF,)),
            pltpu.SemaphoreType.DMA((NBUF,)),
        ],
        compiler_params=pltpu.CompilerParams(
            needs_layout_passes=False, use_tc_tiling_on_sc=False),
    )
    def k(table_hbm, idx_hbm, out_hbm, idx_v, rows_v, gsem, ssem):
        wid = lax.axis_index("s") * nc + lax.axis_index("c")
        row_base = wid * per_w
        pltpu.sync_copy(idx_hbm.at[pl.ds(wid * idx_rows_per_w, idx_rows_per_w)],
                        idx_v)

        def fire_gather(c, b):
            for g in range(GPC):
                pltpu.async_copy(
                    table_hbm.at[idx_v.at[c * GPC + g]],
                    rows_v.at[b, pl.ds(g * G, G)],
                    gsem.at[b],
                )

        def drain_gather(c, b):
            for g in range(GPC):
                pltpu.make_async_copy(
                    table_hbm.at[idx_v.at[c * GPC + g]],
                    rows_v.at[b, pl.ds(g * G, G)],
                    gsem.at[b],
                ).wait()

        def start_store(c, b):
            pltpu.async_copy(
                rows_v.at[b],
                out_hbm.at[pl.ds(row_base + c * CHUNK, CHUNK)],
                ssem.at[b])

        def wait_store(c, b):
            pltpu.make_async_copy(
                rows_v.at[b],
                out_hbm.at[pl.ds(row_base + c * CHUNK, CHUNK)],
                ssem.at[b]).wait()

        fire_gather(0, 0)
        fire_gather(1, 1)

        def outer_body(o, _):
            for j in range(NBUF):
                c = o * NBUF + j
                b = j

                @pl.when(c >= 2)
                def _():
                    wait_store(c - 2, (b - 2) % NBUF)

                @pl.when(c + 2 < n_chunks)
                def _():
                    fire_gather(c + 2, (b + 2) % NBUF)

                drain_gather(c, b)
                _normalize_chunk(rows_v, b)
                start_store(c, b)
            return ()

        lax.fori_loop(0, n_chunks // NBUF, outer_body, (), unroll=False)
        wait_store(n_chunks - 2, (n_chunks - 2) % NBUF)
        wait_store(n_chunks - 1, (n_chunks - 1) % NBUF)

    return k(table, idx2d)


def kernel(x, table):
    b, h = x.shape
    n = b * h
    idx2d = x.reshape(n // G, G).astype(jnp.int32)
    out = _sc_embed_norm(table, idx2d, n_rows=n)
    return out.reshape(b, h, D)
